# baseline (speedup 1.0000x reference)
import jax
import jax.numpy as jnp
from jax.experimental import pallas as pl
from jax.experimental.pallas import tpu as pltpu

_N_BLK = 65536


def _dot_t(w_ref, h):
    return jax.lax.dot_general(
        w_ref[...].astype(jnp.bfloat16), h, (((0,), (0,)), ((), ())),
        preferred_element_type=jnp.float32)


def _mlp_kernel(x_ref, w0_ref, w1_ref, w2_ref, w3_ref, w4_ref, w5_ref,
                w6_ref, c0_ref, c1_ref, c2_ref, c3_ref, c4_ref, out_ref):
    bf16 = jnp.bfloat16
    x = x_ref[...].astype(bf16)
    ha = jnp.maximum(
        (_dot_t(w0_ref, x[0:4]) + c0_ref[0:32]).astype(bf16), 0)
    hb = jnp.maximum(
        (_dot_t(w2_ref, x[4:8]) + c0_ref[32:64]).astype(bf16), 0)
    ha = jnp.maximum(
        (_dot_t(w1_ref, ha) + c1_ref[0:32]).astype(bf16), 0)
    hb = jnp.maximum(
        (_dot_t(w3_ref, hb) + c1_ref[32:64]).astype(bf16), 0)
    h = jnp.concatenate([ha, hb], axis=0)
    h = jnp.maximum((_dot_t(w4_ref, h) + c2_ref[...]).astype(bf16), 0)
    h = jnp.maximum((_dot_t(w5_ref, h) + c3_ref[...]).astype(bf16), 0)
    out_ref[...] = _dot_t(w6_ref, h) + c4_ref[...]


def kernel(x, w0, b0, w1, b1, w2, b2, w3, b3, w4, b4, w5, b5, w6, b6):
    B, D = x.shape
    cs = (jnp.concatenate([b0, b2])[:, None],
          jnp.concatenate([b1, b3])[:, None],
          b4[:, None], b5[:, None], b6[:, None])
    ws = (w0, w1, w2, w3, w4, w5, w6)

    xt = x.T
    b_pad = ((B + _N_BLK - 1) // _N_BLK) * _N_BLK
    if b_pad != B:
        xt = jnp.zeros((D, b_pad), xt.dtype).at[:, :B].set(xt)

    grid = (b_pad // _N_BLK,)
    cost = pl.CostEstimate(
        flops=2 * 8000 * b_pad,
        transcendentals=0,
        bytes_accessed=4 * 16 * b_pad,
    )
    wspecs = [pl.BlockSpec(w.shape, lambda i: (0, 0)) for w in ws + cs]
    out = pl.pallas_call(
        _mlp_kernel,
        out_shape=jax.ShapeDtypeStruct((8, b_pad), jnp.float32),
        grid=grid,
        in_specs=[pl.BlockSpec((8, _N_BLK), lambda i: (0, i))] + wspecs,
        out_specs=pl.BlockSpec((8, _N_BLK), lambda i: (0, i)),
        compiler_params=pltpu.CompilerParams(
            dimension_semantics=("parallel",),
            allow_input_fusion=[False] + [True] * (len(ws) + len(cs)),
        ),
        cost_estimate=cost,
    )(xt, *ws, *cs)

    return out[:, :B].T
